# Initial kernel scaffold; baseline (speedup 1.0000x reference)
import functools

import jax
import jax.numpy as jnp
from jax import lax
from jax.experimental import pallas as pl
from jax.experimental.pallas import tpu as pltpu
from jax.experimental.pallas import tpu_sc as plsc

EMB = 64
F0, F1, F2 = 5, 6, 2
NROWS = F0 * F1 * F2
N_EDGES = 800000
LANES = 16

_info = plsc.get_sparse_core_info()
NC = _info.num_cores
NS = _info.num_subcores
NW = NC * NS
PER_W = N_EDGES // NW
CHUNK = 128
N_FULL = PER_W // CHUNK
TAIL = PER_W - N_FULL * CHUNK

_mesh = plsc.VectorSubcoreMesh(core_axis_name="c", subcore_axis_name="s")


@functools.partial(
    pl.kernel,
    mesh=_mesh,
    out_type=jax.ShapeDtypeStruct((NROWS, EMB), jnp.float32),
    scratch_types=[
        pltpu.VMEM((F0, EMB), jnp.float32),
        pltpu.VMEM((F1, EMB), jnp.float32),
        pltpu.VMEM((F2, EMB), jnp.float32),
        pltpu.VMEM((NROWS, EMB), jnp.float32),
    ],
)
def _build_table(w0_hbm, w1_hbm, w2_hbm, t_hbm, w0_v, w1_v, w2_v, t_v):
    wid = lax.axis_index("s") * NC + lax.axis_index("c")

    @pl.when(wid == 0)
    def _():
        pltpu.sync_copy(w0_hbm, w0_v)
        pltpu.sync_copy(w1_hbm, w1_v)
        pltpu.sync_copy(w2_hbm, w2_v)
        for i in range(F0):
            for j in range(F1):
                for g in range(EMB // LANES):
                    sl = pl.ds(g * LANES, LANES)
                    s01 = w0_v[i, sl] + w1_v[j, sl]
                    for k in range(F2):
                        t_v[(i * F1 + j) * F2 + k, sl] = s01 + w2_v[k, sl]
        pltpu.sync_copy(t_v, t_hbm)


@functools.partial(
    pl.kernel,
    mesh=_mesh,
    out_type=jax.ShapeDtypeStruct((N_EDGES, EMB), jnp.float32),
    scratch_types=[
        pltpu.VMEM((CHUNK, 3), jnp.int32),
        pltpu.VMEM((CHUNK,), jnp.int32),
        pltpu.VMEM((CHUNK, EMB), jnp.float32),
        pltpu.SemaphoreType.DMA,
    ],
)
def _lookup(ea_hbm, t_hbm, out_hbm, ea_v, idx_v, rows_v, sem):
    wid = lax.axis_index("s") * NC + lax.axis_index("c")
    wbase = wid * PER_W
    lane = lax.iota(jnp.int32, LANES)
    col0 = jnp.zeros((LANES,), jnp.int32)
    col1 = col0 + 1
    col2 = col0 + 2

    def fused_index(rid):
        a0 = plsc.load_gather(ea_v, [rid, col0])
        a1 = plsc.load_gather(ea_v, [rid, col1])
        a2 = plsc.load_gather(ea_v, [rid, col2])
        c = a0 * (F1 * F2) + a1 * F2 + a2
        return jnp.minimum(jnp.maximum(c, 0), NROWS - 1)

    def body(t, carry):
        base = wbase + t * CHUNK
        pltpu.sync_copy(ea_hbm.at[pl.ds(base, CHUNK)], ea_v)
        for g in range(CHUNK // LANES):
            idx_v[pl.ds(g * LANES, LANES)] = fused_index(lane + g * LANES)
        pltpu.async_copy(t_hbm.at[idx_v], rows_v, sem).wait()
        pltpu.sync_copy(rows_v, out_hbm.at[pl.ds(base, CHUNK)])
        return carry

    lax.fori_loop(0, N_FULL, body, 0)

    tbase = wbase + N_FULL * CHUNK
    pltpu.sync_copy(ea_hbm.at[pl.ds(tbase, TAIL)], ea_v.at[pl.ds(0, TAIL)])
    for g in range((TAIL + LANES - 1) // LANES):
        rid = jnp.minimum(lane + g * LANES, TAIL - 1)
        idx_v[pl.ds(g * LANES, LANES)] = fused_index(rid)
    pltpu.async_copy(t_hbm.at[idx_v], rows_v, sem).wait()
    pltpu.sync_copy(rows_v.at[pl.ds(0, TAIL)], out_hbm.at[pl.ds(tbase, TAIL)])


def kernel(edge_attr, W0, W1, W2):
    ea = edge_attr.astype(jnp.int32)
    t = _build_table(W0, W1, W2)
    return _lookup(ea, t)


# speedup vs baseline: 1.2891x; 1.2891x over previous
import functools

import jax
import jax.numpy as jnp
from jax import lax
from jax.experimental import pallas as pl
from jax.experimental.pallas import tpu as pltpu
from jax.experimental.pallas import tpu_sc as plsc

EMB = 64
F0, F1, F2 = 5, 6, 2
NROWS = F0 * F1 * F2
NPROWS = NROWS * NROWS
N_EDGES = 800000
N_PAIRS = N_EDGES // 2
LANES = 16

_info = plsc.get_sparse_core_info()
NC = _info.num_cores
NS = _info.num_subcores
NW = NC * NS
CHUNK = 128

PER_W = 12504
N_FULL_A, N_FULL_B = 97, 96
TAIL = 88
assert 31 * PER_W + N_FULL_B * CHUNK + TAIL == N_PAIRS
assert PER_W % 8 == 0 and TAIL % 8 == 0

_mesh = plsc.VectorSubcoreMesh(core_axis_name="c", subcore_axis_name="s")


def _pair_table_body(w0_ref, w1_ref, w2_ref, out_ref):
    r = lax.broadcasted_iota(jnp.int32, (NROWS, F0), 0)
    c = lax.broadcasted_iota(jnp.int32, (NROWS, F0), 1)
    o0 = (r // (F1 * F2) == c).astype(jnp.float32)
    r = lax.broadcasted_iota(jnp.int32, (NROWS, F1), 0)
    c = lax.broadcasted_iota(jnp.int32, (NROWS, F1), 1)
    o1 = ((r // F2) % F1 == c).astype(jnp.float32)
    r = lax.broadcasted_iota(jnp.int32, (NROWS, F2), 0)
    c = lax.broadcasted_iota(jnp.int32, (NROWS, F2), 1)
    o2 = (r % F2 == c).astype(jnp.float32)
    t = (jnp.dot(o0, w0_ref[...], preferred_element_type=jnp.float32, precision=lax.Precision.HIGHEST)
         + jnp.dot(o1, w1_ref[...], preferred_element_type=jnp.float32, precision=lax.Precision.HIGHEST)
         + jnp.dot(o2, w2_ref[...], preferred_element_type=jnp.float32, precision=lax.Precision.HIGHEST))
    rp = lax.broadcasted_iota(jnp.int32, (NPROWS, NROWS), 0)
    cp = lax.broadcasted_iota(jnp.int32, (NPROWS, NROWS), 1)
    p1 = (rp // NROWS == cp).astype(jnp.float32)
    p2 = (rp % NROWS == cp).astype(jnp.float32)
    out_ref[:, :EMB] = jnp.dot(p1, t, preferred_element_type=jnp.float32, precision=lax.Precision.HIGHEST)
    out_ref[:, EMB:] = jnp.dot(p2, t, preferred_element_type=jnp.float32, precision=lax.Precision.HIGHEST)


_pair_table = pl.pallas_call(
    _pair_table_body,
    out_shape=jax.ShapeDtypeStruct((NPROWS, 2 * EMB), jnp.float32),
)


@functools.partial(
    pl.kernel,
    mesh=_mesh,
    out_type=jax.ShapeDtypeStruct((N_PAIRS, 2 * EMB), jnp.float32),
    scratch_types=[
        pltpu.VMEM((CHUNK,), jnp.int32),
        pltpu.VMEM((CHUNK,), jnp.int32),
        pltpu.VMEM((CHUNK,), jnp.int32),
        pltpu.VMEM((CHUNK,), jnp.int32),
        pltpu.VMEM((CHUNK,), jnp.int32),
        pltpu.VMEM((CHUNK,), jnp.int32),
        pltpu.VMEM((CHUNK,), jnp.int32),
        pltpu.VMEM((CHUNK, 2 * EMB), jnp.float32),
        pltpu.SemaphoreType.DMA,
    ],
)
def _lookup(c0_hbm, c1_hbm, c2_hbm, c3_hbm, c4_hbm, c5_hbm, tp_hbm, out_hbm,
            c0_v, c1_v, c2_v, c3_v, c4_v, c5_v, idx_v, rows_v, sem):
    wid = lax.axis_index("s") * NC + lax.axis_index("c")
    wbase = wid * PER_W
    n_full = jnp.where(wid == NW - 1, N_FULL_B, N_FULL_A)

    def compute_indices():
        for g in range(CHUNK // LANES):
            sl = pl.ds(g * LANES, LANES)
            ca = c0_v[sl] * (F1 * F2) + c1_v[sl] * F2 + c2_v[sl]
            cb = c3_v[sl] * (F1 * F2) + c4_v[sl] * F2 + c5_v[sl]
            c = ca * NROWS + cb
            idx_v[sl] = jnp.minimum(jnp.maximum(c, 0), NPROWS - 1)

    def load_cols(base, size):
        pltpu.sync_copy(c0_hbm.at[pl.ds(base, size)], c0_v.at[pl.ds(0, size)])
        pltpu.sync_copy(c1_hbm.at[pl.ds(base, size)], c1_v.at[pl.ds(0, size)])
        pltpu.sync_copy(c2_hbm.at[pl.ds(base, size)], c2_v.at[pl.ds(0, size)])
        pltpu.sync_copy(c3_hbm.at[pl.ds(base, size)], c3_v.at[pl.ds(0, size)])
        pltpu.sync_copy(c4_hbm.at[pl.ds(base, size)], c4_v.at[pl.ds(0, size)])
        pltpu.sync_copy(c5_hbm.at[pl.ds(base, size)], c5_v.at[pl.ds(0, size)])

    def body(t, carry):
        base = wbase + t * CHUNK
        load_cols(base, CHUNK)
        compute_indices()
        pltpu.async_copy(tp_hbm.at[idx_v], rows_v, sem).wait()
        pltpu.sync_copy(rows_v, out_hbm.at[pl.ds(base, CHUNK)])
        return carry

    lax.fori_loop(0, n_full, body, 0)

    tbase = wbase + n_full * CHUNK
    load_cols(tbase, TAIL)
    compute_indices()
    pltpu.async_copy(tp_hbm.at[idx_v], rows_v, sem).wait()
    pltpu.sync_copy(rows_v.at[pl.ds(0, TAIL)], out_hbm.at[pl.ds(tbase, TAIL)])


def kernel(edge_attr, W0, W1, W2):
    eap = edge_attr.astype(jnp.int32).reshape(N_PAIRS, 6)
    cols = [eap[:, i] for i in range(6)]
    tp = _pair_table(W0, W1, W2)
    out = _lookup(*cols, tp)
    return out.reshape(N_EDGES, EMB)
